# baseline (speedup 1.0000x reference)
import functools

import jax
import jax.numpy as jnp
from jax import lax
from jax.experimental import pallas as pl
from jax.experimental.pallas import tpu as pltpu
from jax.experimental.pallas import tpu_sc as plsc

_B = 16384
_D = 64
_K = 4
_ROW = _K * _D
_HALF = 128
_NW = 32
_BPW = _B // _NW
_CH = 128
_NCH = _BPW // _CH
_G = 16

_mesh = plsc.VectorSubcoreMesh(core_axis_name="c", subcore_axis_name="s")


@functools.partial(
    pl.kernel,
    out_type=jax.ShapeDtypeStruct((_B,), jnp.float32),
    mesh=_mesh,
    compiler_params=pltpu.CompilerParams(
        needs_layout_passes=False, use_tc_tiling_on_sc=False),
    scratch_types=[
        pltpu.VMEM((_BPW,), jnp.int32),
        pltpu.VMEM((2, _CH), jnp.int32),
        pltpu.VMEM((2, _CH), jnp.int32),
        pltpu.VMEM((_BPW * _D,), jnp.float32),
        pltpu.VMEM((2, _CH, _HALF), jnp.float32),
        pltpu.VMEM((2, _CH, _HALF), jnp.float32),
        pltpu.VMEM((_BPW,), jnp.float32),
        pltpu.SemaphoreType.DMA,
        pltpu.SemaphoreType.DMA,
        pltpu.SemaphoreType.DMA,
        pltpu.SemaphoreType.DMA,
    ],
)
def _sc_codebook(codes_hbm, pred_hbm, cent_hbm, out_hbm,
                 idx_v, idxe_v, idxo_v, codes_v, rowse_v, rowso_v, out_v,
                 seme0, seme1, semo0, semo1):
    wid = lax.axis_index("s") * 2 + lax.axis_index("c")
    base = pl.multiple_of(wid * _BPW, _BPW)

    pltpu.sync_copy(pred_hbm.at[pl.ds(base, _BPW)], idx_v)
    pltpu.sync_copy(codes_hbm.at[pl.ds(base * _D, _BPW * _D)], codes_v)

    seme = (seme0, seme1)
    semo = (semo0, semo1)
    lane = lax.iota(jnp.int32, _G)

    def start_gather(ch):
        p = ch % 2
        for g in range(_CH // _G):
            c16 = idx_v[pl.ds(ch * _CH + g * _G, _G)]
            idxe_v[p, pl.ds(g * _G, _G)] = c16 * 2
            idxo_v[p, pl.ds(g * _G, _G)] = c16 * 2 + 1
        cpe = pltpu.make_async_copy(
            cent_hbm.at[idxe_v.at[p]], rowse_v.at[p], seme[p])
        cpo = pltpu.make_async_copy(
            cent_hbm.at[idxo_v.at[p]], rowso_v.at[p], semo[p])
        cpe.start()
        cpo.start()
        return (cpe, cpo)

    inv_d = jnp.float32(1.0 / _D)
    lane0 = lane == 0

    def compute_chunk(ch):
        p = ch % 2
        rowse = rowse_v.at[p]
        rowso = rowso_v.at[p]

        def item_body(i, _):
            cvecs = [codes_v[pl.ds((ch * _CH + i) * _D + j * _G, _G)]
                     for j in range(_D // _G)]
            sums = []
            for half, rows in ((0, rowse), (1, rowso)):
                for kk in range(2):
                    acc = None
                    for j in range(_D // _G):
                        cent = rows[i, pl.ds(kk * _D + j * _G, _G)]
                        term = jnp.abs(cvecs[j] - cent)
                        acc = term if acc is None else acc + term
                    sums.append(jnp.sum(acc))
            m = jnp.minimum(jnp.minimum(sums[0], sums[1]),
                            jnp.minimum(sums[2], sums[3]))
            plsc.store_scatter(
                out_v,
                [jnp.broadcast_to(ch * _CH + i, (_G,))],
                jnp.broadcast_to(m * inv_d, (_G,)),
                mask=lane0,
            )
            return 0

        lax.fori_loop(0, _CH, item_body, 0)

    descs = [None, None]
    descs[0] = start_gather(0)
    for ch in range(_NCH):
        if ch + 1 < _NCH:
            descs[(ch + 1) % 2] = start_gather(ch + 1)
        descs[ch % 2][0].wait()
        descs[ch % 2][1].wait()
        compute_chunk(ch)

    pltpu.sync_copy(out_v, out_hbm.at[pl.ds(base, _BPW)])


def kernel(codes, pred_class, centroids):
    cent2 = centroids.reshape(2 * centroids.shape[0], _HALF)
    codes_flat = codes.reshape(_B * _D)
    pred = pred_class.astype(jnp.int32)
    return _sc_codebook(codes_flat, pred, cent2)


# speedup vs baseline: 1.5159x; 1.5159x over previous
import functools

import jax
import jax.numpy as jnp
from jax import lax
from jax.experimental import pallas as pl
from jax.experimental.pallas import tpu as pltpu
from jax.experimental.pallas import tpu_sc as plsc

_B = 16384
_D = 64
_K = 4
_ROW = _K * _D
_HALF = 128
_NW = 32
_BPW = _B // _NW
_CH = 128
_NCH = _BPW // _CH
_G = 16

_mesh = plsc.VectorSubcoreMesh(core_axis_name="c", subcore_axis_name="s")


@functools.partial(
    pl.kernel,
    out_type=jax.ShapeDtypeStruct((_B,), jnp.float32),
    mesh=_mesh,
    compiler_params=pltpu.CompilerParams(
        needs_layout_passes=False, use_tc_tiling_on_sc=False),
    scratch_types=[
        pltpu.VMEM((_BPW,), jnp.int32),
        pltpu.VMEM((_BPW * _D,), jnp.float32),
        pltpu.VMEM((2, _CH, _ROW), jnp.float32),
        pltpu.VMEM((_BPW,), jnp.float32),
        pltpu.SemaphoreType.DMA,
        pltpu.SemaphoreType.DMA,
    ],
)
def _sc_codebook(codes_hbm, pred_hbm, cent_hbm, out_hbm,
                 idx_v, codes_v, rows_v, out_v, sem0, sem1):
    wid = lax.axis_index("s") * 2 + lax.axis_index("c")
    base = pl.multiple_of(wid * _BPW, _BPW)

    pltpu.sync_copy(pred_hbm.at[pl.ds(base, _BPW)], idx_v)
    pltpu.sync_copy(codes_hbm.at[pl.ds(base * _D, _BPW * _D)], codes_v)

    sems = (sem0, sem1)
    lane = lax.iota(jnp.int32, _G)

    def start_gather(ch):
        p = ch % 2
        cp = pltpu.make_async_copy(
            cent_hbm.at[idx_v.at[pl.ds(ch * _CH, _CH)]],
            rows_v.at[p], sems[p])
        cp.start()
        return cp

    inv_d = jnp.float32(1.0 / _D)
    lane0 = lane == 0

    def compute_chunk(ch):
        p = ch % 2
        rows = rows_v.at[p]

        def item_body(i, _):
            cvecs = [codes_v[pl.ds((ch * _CH + i) * _D + j * _G, _G)]
                     for j in range(_D // _G)]
            sums = []
            for k in range(_K):
                acc = None
                for j in range(_D // _G):
                    cent = rows[i, pl.ds(k * _D + j * _G, _G)]
                    term = jnp.abs(cvecs[j] - cent)
                    acc = term if acc is None else acc + term
                sums.append(jnp.sum(acc))
            m = jnp.minimum(jnp.minimum(sums[0], sums[1]),
                            jnp.minimum(sums[2], sums[3]))
            plsc.store_scatter(
                out_v,
                [jnp.broadcast_to(ch * _CH + i, (_G,))],
                jnp.broadcast_to(m * inv_d, (_G,)),
                mask=lane0,
            )
            return 0

        lax.fori_loop(0, _CH, item_body, 0)

    descs = [None, None]
    descs[0] = start_gather(0)
    for ch in range(_NCH):
        if ch + 1 < _NCH:
            descs[(ch + 1) % 2] = start_gather(ch + 1)
        descs[ch % 2].wait()
        compute_chunk(ch)

    pltpu.sync_copy(out_v, out_hbm.at[pl.ds(base, _BPW)])


def kernel(codes, pred_class, centroids):
    cent2 = centroids.reshape(centroids.shape[0], _ROW)
    codes_flat = codes.reshape(_B * _D)
    pred = pred_class.astype(jnp.int32)
    return _sc_codebook(codes_flat, pred, cent2)
